# Initial kernel scaffold; baseline (speedup 1.0000x reference)
import jax
import jax.numpy as jnp
from jax.experimental import pallas as pl


def kernel(x, edge_index, Wp, bp, Wa, ba, Wpair, Wc1, bc1, ln_g, ln_b, Wc2, bc2, Wt):
    raise NotImplementedError("write your pallas kernel here")


# speedup vs baseline: 5.9997x; 5.9997x over previous
import functools

import jax
import jax.numpy as jnp
from jax import lax
from jax.experimental import pallas as pl
from jax.experimental.pallas import tpu as pltpu
from jax.experimental.pallas import tpu_sc as plsc

N = 10000
E = 320000
IN = 128
H = 128
CH = 64
L = 4
TAU0 = 1.0

NW = 32
CHUNK = 128
CPW = 79
NCH = NW * CPW
EPAD = NCH * CHUNK
NR = 10112
RPS = NR // 16

BLK = 512
GRID = (N + BLK - 1) // BLK


def _sc_mesh():
    return plsc.VectorSubcoreMesh(core_axis_name="c", subcore_axis_name="s",
                                  num_cores=2, num_subcores=16)


def _mm_t(a, b):
    return lax.dot_general(a, b, (((1,), (1,)), ((), ())))


def _colp_body(row_ref, col_ref, colp_ref):
    r = row_ref[...]
    c = col_ref[...]
    colp_ref[...] = jnp.where(r == c, N, c)


def _colp_call(row2d, col2d):
    return pl.pallas_call(
        _colp_body,
        out_shape=jax.ShapeDtypeStruct((NCH, CHUNK), jnp.int32),
    )(row2d, col2d)


def _deg_body(colp_hbm, deg_hbm, cidx_v, ones_v, zb_v, deg_sh):
    cid = lax.axis_index("c")
    sid = lax.axis_index("s")
    wid = cid * 16 + sid

    def init_bufs(i, _):
        for j in range(H // 16):
            ones_v[i, pl.ds(j * 16, 16)] = jnp.full((16,), 1.0, jnp.float32)
            zb_v[i, pl.ds(j * 16, 16)] = jnp.zeros((16,), jnp.float32)
        return 0
    lax.fori_loop(0, CHUNK, init_bufs, 0)
    for t in range(RPS // CHUNK):
        pltpu.sync_copy(zb_v, deg_sh.at[pl.ds(sid * RPS + t * CHUNK, CHUNK)])
    rem = RPS % CHUNK
    pltpu.sync_copy(zb_v.at[pl.ds(0, rem)],
                    deg_sh.at[pl.ds(sid * RPS + (RPS - rem), rem)])
    plsc.subcore_barrier()

    def body(k, _):
        ci = wid * CPW + k
        pltpu.sync_copy(colp_hbm.at[ci], cidx_v)
        pltpu.sync_copy(ones_v, deg_sh.at[cidx_v], add=True)
        return 0
    lax.fori_loop(0, CPW, body, 0)
    plsc.subcore_barrier()
    pltpu.sync_copy(deg_sh.at[pl.ds(sid * RPS, RPS)],
                    deg_hbm.at[cid, pl.ds(sid * RPS, RPS)])


@functools.lru_cache(maxsize=1)
def _deg_kernel():
    return pl.kernel(
        _deg_body,
        out_type=jax.ShapeDtypeStruct((2, NR, H), jnp.float32),
        mesh=_sc_mesh(),
        scratch_types=[
            pltpu.VMEM((CHUNK,), jnp.int32),
            pltpu.VMEM((CHUNK, H), jnp.float32),
            pltpu.VMEM((CHUNK, H), jnp.float32),
            pltpu.VMEM_SHARED((NR, H), jnp.float32),
        ],
    )


def _deg_call(colp2d):
    return _deg_kernel()(colp2d)


def _agg_body(out_hbm, row_hbm, colp_hbm, part_hbm,
              ridx_v, cidx_v, rows_v, sem, agg_sh):
    cid = lax.axis_index("c")
    sid = lax.axis_index("s")
    wid = cid * 16 + sid

    def zb(i, _):
        for j in range(H // 16):
            rows_v[i, pl.ds(j * 16, 16)] = jnp.zeros((16,), jnp.float32)
        return 0
    lax.fori_loop(0, CHUNK, zb, 0)
    for t in range(RPS // CHUNK):
        pltpu.sync_copy(rows_v, agg_sh.at[pl.ds(sid * RPS + t * CHUNK, CHUNK)])
    rem = RPS % CHUNK
    pltpu.sync_copy(rows_v.at[pl.ds(0, rem)],
                    agg_sh.at[pl.ds(sid * RPS + (RPS - rem), rem)])
    plsc.subcore_barrier()

    def body(k, _):
        ci = wid * CPW + k
        pltpu.sync_copy(row_hbm.at[ci], ridx_v)
        pltpu.sync_copy(colp_hbm.at[ci], cidx_v)
        pltpu.async_copy(out_hbm.at[ridx_v], rows_v, sem).wait()
        pltpu.sync_copy(rows_v, agg_sh.at[cidx_v], add=True)
        return 0
    lax.fori_loop(0, CPW, body, 0)
    plsc.subcore_barrier()
    pltpu.sync_copy(agg_sh.at[pl.ds(sid * RPS, RPS)],
                    part_hbm.at[cid, pl.ds(sid * RPS, RPS)])


@functools.lru_cache(maxsize=1)
def _agg_kernel():
    return pl.kernel(
        _agg_body,
        out_type=jax.ShapeDtypeStruct((2, NR, H), jnp.float32),
        mesh=_sc_mesh(),
        scratch_types=[
            pltpu.VMEM((CHUNK,), jnp.int32),
            pltpu.VMEM((CHUNK,), jnp.int32),
            pltpu.VMEM((CHUNK, H), jnp.float32),
            pltpu.SemaphoreType.DMA,
            pltpu.VMEM_SHARED((NR, H), jnp.float32),
        ],
    )


def _agg_call(out, row2d, colp2d):
    return _agg_kernel()(out, row2d, colp2d)


def _node_spec(d):
    return pl.BlockSpec((BLK, d), lambda i: (i, 0))


def _full_spec(shape):
    return pl.BlockSpec(shape, lambda i: (0,) * len(shape))


def _proj_body(x_ref, wp_ref, bp_ref, pre_ref):
    pre_ref[...] = _mm_t(x_ref[...], wp_ref[...]) + bp_ref[...]


def _proj_call(x, wp, bp2):
    return pl.pallas_call(
        _proj_body,
        grid=(GRID,),
        in_specs=[_node_spec(IN), _full_spec((H, IN)), _full_spec((1, H))],
        out_specs=[_node_spec(H)],
        out_shape=(jax.ShapeDtypeStruct((N, H), jnp.float32),),
    )(x, wp, bp2)[0]


def _mm4_body(h_ref, dis_ref, wc1_ref, bc1_ref, wt_ref, wsym_ref, anti_ref,
              ba_ref, prec_ref, pret_ref, out_ref, ha_ref):
    h = h_ref[...]
    prec_ref[...] = _mm_t(h, wc1_ref[...]) + bc1_ref[...]
    pret_ref[...] = _mm_t(h, wt_ref[...])
    out_ref[...] = _mm_t(h, wsym_ref[...]) * dis_ref[...]
    ha_ref[...] = _mm_t(h, anti_ref[...]) + ba_ref[...]


def _mm4_call(h, dis, wc1, bc12, wt, wsym, anti, ba2):
    return pl.pallas_call(
        _mm4_body,
        grid=(GRID,),
        in_specs=[
            _node_spec(H), _node_spec(1),
            _full_spec((CH, H)), _full_spec((1, CH)),
            _full_spec((1, H)), _full_spec((H, H)), _full_spec((H, H)),
            _full_spec((1, H)),
        ],
        out_specs=[_node_spec(CH), _node_spec(1), _node_spec(H),
                   _node_spec(H)],
        out_shape=(
            jax.ShapeDtypeStruct((N, CH), jnp.float32),
            jax.ShapeDtypeStruct((N, 1), jnp.float32),
            jax.ShapeDtypeStruct((N, H), jnp.float32),
            jax.ShapeDtypeStruct((N, H), jnp.float32),
        ),
    )(h, dis, wc1, bc12, wt, wsym, anti, ba2)


def _logits_body(h1n_ref, wc2_ref, bc2_ref, lg_ref):
    lg_ref[...] = _mm_t(h1n_ref[...], wc2_ref[...]) + bc2_ref[...]


def _logits_call(h1n, wc2, bc22):
    return pl.pallas_call(
        _logits_body,
        grid=(GRID,),
        in_specs=[_node_spec(CH), _full_spec((2, CH)), _full_spec((1, 2))],
        out_specs=[_node_spec(2)],
        out_shape=(jax.ShapeDtypeStruct((N, 2), jnp.float32),),
    )(h1n, wc2, bc22)[0]


def kernel(x, edge_index, Wp, bp, Wa, ba, Wpair, Wc1, bc1, ln_g, ln_b,
           Wc2, bc2, Wt):
    row = edge_index[0]
    col = edge_index[1]
    pad = EPAD - E
    row2d = jnp.concatenate(
        [row, jnp.zeros((pad,), jnp.int32)]).reshape(NCH, CHUNK)
    col2d = jnp.concatenate(
        [col, jnp.full((pad,), N, jnp.int32)]).reshape(NCH, CHUNK)

    W0 = jnp.triu(Wpair[:, :-2], 1)
    W0 = W0 + W0.T
    q = Wpair[:, -2]
    r = Wpair[:, -1]
    wsym = W0 + jnp.diag(q * jnp.sum(jnp.abs(W0), axis=1) + r)
    anti = Wa - Wa.T

    bp2 = bp.reshape(1, H)
    ba2 = ba.reshape(1, H)
    bc12 = bc1.reshape(1, CH)
    bc22 = bc2.reshape(1, 2)
    wt2 = Wt.reshape(1, H)

    colp2d = _colp_call(row2d, col2d)
    degp = _deg_call(colp2d)
    deg = degp[0, :N, 0] + degp[1, :N, 0]
    dis = jnp.where(deg > 0, deg ** -0.5, 0.0)
    dis2 = dis.reshape(N, 1)

    h = jax.nn.gelu(_proj_call(x, Wp, bp2), approximate=False)

    z = jnp.zeros_like(h)
    cont = jnp.ones((N,), dtype=bool)
    step = jnp.ones((N, 1), dtype=h.dtype)
    exit_layers = jnp.full((N,), L, dtype=jnp.int32)
    nkey = jax.random.key(42)
    for li in range(L):
        pre_c, pre_t, out, ha = _mm4_call(h, dis2, Wc1, bc12, wt2, wsym,
                                          anti, ba2)
        h1 = jax.nn.gelu(pre_c, approximate=False)
        mu = jnp.mean(h1, axis=-1, keepdims=True)
        var = jnp.mean((h1 - mu) ** 2, axis=-1, keepdims=True)
        h1n = (h1 - mu) / jnp.sqrt(var + 1e-5) * ln_g + ln_b
        logits = _logits_call(h1n, Wc2, bc22)
        temp = 1.0 / (jax.nn.softplus(pre_t) + TAU0)
        u = jax.random.uniform(jax.random.fold_in(nkey, li), (N, 2),
                               minval=1e-10, maxval=1.0)
        g = -jnp.log(-jnp.log(u))
        gum = (logits + g) / temp
        ys = jax.nn.softmax(gum, axis=-1)
        yh = jax.nn.one_hot(jnp.argmax(ys, axis=-1), 2, dtype=ys.dtype)
        y = yh - jax.lax.stop_gradient(ys) + ys
        step = step * y[:, 0:1]
        ed = y[:, 1] > y[:, 0]
        newly = ed & cont
        z = z + h * newly[:, None].astype(h.dtype)
        exit_layers = jnp.where(newly, li, exit_layers)
        cont = cont & (~ed)
        parts = _agg_call(out, row2d, colp2d)
        agg = (parts[0, :N] + parts[1, :N]) * dis2
        delta = -ha + agg
        h = h + step * jax.nn.gelu(delta, approximate=False)
    z = z + h * cont[:, None].astype(h.dtype)
    return (z, exit_layers)
